# baseline (speedup 1.0000x reference)
import jax
import jax.numpy as jnp
from jax.experimental import pallas as pl
from jax.experimental.pallas import tpu as pltpu

_LANES = 128
_ROWS = 3750
_CORES = 2
_JSTEPS = 25


def _roll_l(x, k):
    return jnp.concatenate([x[:, k:], x[:, :k]], axis=1)


def _iou_body(p_ref, t_ref, sum_ref, cnt_ref):
    j = pl.program_id(1)
    p = p_ref[0]
    t = t_ref[0]

    pwh = _roll_l(p, 2)
    twh = _roll_l(t, 2)
    pa = p - 0.5 * pwh
    pb = p + 0.5 * pwh
    ta = t - 0.5 * twh
    tb = t + 0.5 * twh

    lo = jnp.maximum(pa, ta)
    hi = jnp.minimum(pb, tb)
    d = jnp.maximum(hi - lo, 0.0)
    inter = d * _roll_l(d, 1)

    wp = pb - pa
    wt = tb - ta
    area_p = jnp.abs(wp * _roll_l(wp, 1))
    area_t = jnp.abs(wt * _roll_l(wt, 1))

    iou = inter / (area_p + area_t - inter + 1e-6)

    lane = jax.lax.broadcasted_iota(jnp.int32, p.shape, 1)
    valid = ((lane & 3) == 0) & (t != -1.0)
    vf = jnp.where(valid, 1.0, 0.0)
    contrib = jnp.where(valid, iou, 0.0)

    s = jnp.sum(contrib, axis=0, keepdims=True)
    c = jnp.sum(vf, axis=0, keepdims=True)

    @pl.when(j == 0)
    def _():
        sum_ref[...] = jnp.zeros_like(sum_ref)
        cnt_ref[...] = jnp.zeros_like(cnt_ref)

    sum_ref[0] += s
    cnt_ref[0] += c


def kernel(pred, truth):
    total = pred.size
    assert total == _CORES * _JSTEPS * _ROWS * _LANES
    p3 = pred.reshape(_CORES * _JSTEPS, _ROWS, _LANES)
    t3 = truth.reshape(_CORES * _JSTEPS, _ROWS, _LANES)

    in_spec = pl.BlockSpec(
        (1, _ROWS, _LANES), lambda i, j: (i * _JSTEPS + j, 0, 0)
    )
    out_spec = pl.BlockSpec((1, 1, _LANES), lambda i, j: (i, 0, 0))
    sums, cnts = pl.pallas_call(
        _iou_body,
        grid=(_CORES, _JSTEPS),
        in_specs=[in_spec, in_spec],
        out_specs=[out_spec, out_spec],
        out_shape=[
            jax.ShapeDtypeStruct((_CORES, 1, _LANES), jnp.float32),
            jax.ShapeDtypeStruct((_CORES, 1, _LANES), jnp.float32),
        ],
        compiler_params=pltpu.CompilerParams(
            dimension_semantics=("parallel", "arbitrary"),
        ),
    )(p3, t3)

    total_iou = jnp.sum(sums)
    n_valid = jnp.sum(cnts)
    mean_iou = jnp.where(
        n_valid > 0, total_iou / jnp.maximum(n_valid, 1.0), 0.0
    )
    return mean_iou.reshape(1, 1)


# speedup vs baseline: 154.6262x; 154.6262x over previous
import jax
import jax.numpy as jnp
from jax.experimental import pallas as pl
from jax.experimental.pallas import tpu as pltpu

_B = 1_000_000
_NBOX = 6
_LBLK = 16384
_CORES = 2
_JSTEPS = 31


def _iou_body(p_ref, t_ref, sum_ref, cnt_ref):
    i = pl.program_id(0)
    j = pl.program_id(1)

    s = jnp.zeros((1, _LBLK), jnp.float32)
    c = jnp.zeros((1, _LBLK), jnp.float32)
    lane = jax.lax.broadcasted_iota(jnp.int32, (1, _LBLK), 1)
    base = (i * _JSTEPS + j) * _LBLK
    in_bounds = (base + lane) < _B

    for b in range(_NBOX):
        pcx = p_ref[b, 0:1, :]
        pcy = p_ref[b, 1:2, :]
        pw = p_ref[b, 2:3, :]
        ph = p_ref[b, 3:4, :]
        tcx = t_ref[b, 0:1, :]
        tcy = t_ref[b, 1:2, :]
        tw = t_ref[b, 2:3, :]
        th = t_ref[b, 3:4, :]

        px1 = pcx - 0.5 * pw
        px2 = pcx + 0.5 * pw
        py1 = pcy - 0.5 * ph
        py2 = pcy + 0.5 * ph
        tx1 = tcx - 0.5 * tw
        tx2 = tcx + 0.5 * tw
        ty1 = tcy - 0.5 * th
        ty2 = tcy + 0.5 * th

        dx = jnp.maximum(
            jnp.minimum(px2, tx2) - jnp.maximum(px1, tx1), 0.0
        )
        dy = jnp.maximum(
            jnp.minimum(py2, ty2) - jnp.maximum(py1, ty1), 0.0
        )
        inter = dx * dy
        area_p = jnp.abs((px2 - px1) * (py2 - py1))
        area_t = jnp.abs((tx2 - tx1) * (ty2 - ty1))
        iou = inter / (area_p + area_t - inter + 1e-6)

        valid = (tcx != -1.0) & in_bounds
        s = s + jnp.where(valid, iou, 0.0)
        c = c + jnp.where(valid, 1.0, 0.0)

    @pl.when(j == 0)
    def _():
        sum_ref[...] = jnp.zeros_like(sum_ref)
        cnt_ref[...] = jnp.zeros_like(cnt_ref)

    sum_ref[0] += s
    cnt_ref[0] += c


def kernel(pred, truth):
    p3 = jnp.transpose(pred, (1, 2, 0))
    t3 = jnp.transpose(truth, (1, 2, 0))

    in_spec = pl.BlockSpec(
        (_NBOX, 4, _LBLK), lambda i, j: (0, 0, i * _JSTEPS + j)
    )
    out_spec = pl.BlockSpec((1, 1, _LBLK), lambda i, j: (i, 0, 0))
    sums, cnts = pl.pallas_call(
        _iou_body,
        grid=(_CORES, _JSTEPS),
        in_specs=[in_spec, in_spec],
        out_specs=[out_spec, out_spec],
        out_shape=[
            jax.ShapeDtypeStruct((_CORES, 1, _LBLK), jnp.float32),
            jax.ShapeDtypeStruct((_CORES, 1, _LBLK), jnp.float32),
        ],
        compiler_params=pltpu.CompilerParams(
            dimension_semantics=("parallel", "arbitrary"),
        ),
    )(p3, t3)

    total_iou = jnp.sum(sums)
    n_valid = jnp.sum(cnts)
    mean_iou = jnp.where(
        n_valid > 0, total_iou / jnp.maximum(n_valid, 1.0), 0.0
    )
    return mean_iou.reshape(1, 1)


# speedup vs baseline: 168.9169x; 1.0924x over previous
import jax
import jax.numpy as jnp
from jax.experimental import pallas as pl
from jax.experimental.pallas import tpu as pltpu

_B = 1_000_000
_NBOX = 6
_LBLK = 16384
_CORES = 2
_JSTEPS = 31


_CHUNK = 2048


def _iou_body(p_ref, t_ref, sum_ref, cnt_ref):
    i = pl.program_id(0)
    j = pl.program_id(1)

    lane = jax.lax.broadcasted_iota(jnp.int32, (1, _LBLK), 1)
    base = (i * _JSTEPS + j) * _LBLK
    in_bounds = (base + lane) < _B

    @pl.when(j == 0)
    def _():
        sum_ref[...] = jnp.zeros_like(sum_ref)
        cnt_ref[...] = jnp.zeros_like(cnt_ref)

    for k in range(_LBLK // _CHUNK):
        lo, hi = k * _CHUNK, (k + 1) * _CHUNK
        ib = in_bounds[:, lo:hi]
        sk = jnp.zeros((1, _CHUNK), jnp.float32)
        ck = jnp.zeros((1, _CHUNK), jnp.float32)
        for b in range(_NBOX):
            pcx = p_ref[b, 0:1, lo:hi]
            pcy = p_ref[b, 1:2, lo:hi]
            pw = p_ref[b, 2:3, lo:hi]
            ph = p_ref[b, 3:4, lo:hi]
            tcx = t_ref[b, 0:1, lo:hi]
            tcy = t_ref[b, 1:2, lo:hi]
            tw = t_ref[b, 2:3, lo:hi]
            th = t_ref[b, 3:4, lo:hi]

            px1 = pcx - 0.5 * pw
            px2 = pcx + 0.5 * pw
            py1 = pcy - 0.5 * ph
            py2 = pcy + 0.5 * ph
            tx1 = tcx - 0.5 * tw
            tx2 = tcx + 0.5 * tw
            ty1 = tcy - 0.5 * th
            ty2 = tcy + 0.5 * th

            dx = jnp.maximum(
                jnp.minimum(px2, tx2) - jnp.maximum(px1, tx1), 0.0
            )
            dy = jnp.maximum(
                jnp.minimum(py2, ty2) - jnp.maximum(py1, ty1), 0.0
            )
            inter = dx * dy
            area_p = jnp.abs(pw * ph)
            area_t = jnp.abs(tw * th)
            iou = inter / (area_p + area_t - inter + 1e-6)

            valid = (tcx != -1.0) & ib
            sk = sk + jnp.where(valid, iou, 0.0)
            ck = ck + jnp.where(valid, 1.0, 0.0)
        sum_ref[0, :, lo:hi] += sk
        cnt_ref[0, :, lo:hi] += ck


def kernel(pred, truth):
    p3 = jnp.transpose(pred, (1, 2, 0))
    t3 = jnp.transpose(truth, (1, 2, 0))

    in_spec = pl.BlockSpec(
        (_NBOX, 4, _LBLK), lambda i, j: (0, 0, i * _JSTEPS + j)
    )
    out_spec = pl.BlockSpec((1, 1, _LBLK), lambda i, j: (i, 0, 0))
    sums, cnts = pl.pallas_call(
        _iou_body,
        grid=(_CORES, _JSTEPS),
        in_specs=[in_spec, in_spec],
        out_specs=[out_spec, out_spec],
        out_shape=[
            jax.ShapeDtypeStruct((_CORES, 1, _LBLK), jnp.float32),
            jax.ShapeDtypeStruct((_CORES, 1, _LBLK), jnp.float32),
        ],
        compiler_params=pltpu.CompilerParams(
            dimension_semantics=("parallel", "arbitrary"),
        ),
    )(p3, t3)

    total_iou = jnp.sum(sums)
    n_valid = jnp.sum(cnts)
    mean_iou = jnp.where(
        n_valid > 0, total_iou / jnp.maximum(n_valid, 1.0), 0.0
    )
    return mean_iou.reshape(1, 1)


# speedup vs baseline: 205.3736x; 1.2158x over previous
import jax
import jax.numpy as jnp
from jax.experimental import pallas as pl
from jax.experimental.pallas import tpu as pltpu

_B = 1_000_000
_NBOX = 6
_LBLK = 31744
_CORES = 2
_JSTEPS = 16


_CHUNK = 2048


def _iou_body(p_ref, t_ref, sum_ref, cnt_ref):
    i = pl.program_id(0)
    j = pl.program_id(1)

    lane = jax.lax.broadcasted_iota(jnp.int32, (1, _LBLK), 1)
    base = (i * _JSTEPS + j) * _LBLK
    in_bounds = (base + lane) < _B

    @pl.when(j == 0)
    def _():
        sum_ref[...] = jnp.zeros_like(sum_ref)
        cnt_ref[...] = jnp.zeros_like(cnt_ref)

    for k in range(_LBLK // _CHUNK):
        lo, hi = k * _CHUNK, (k + 1) * _CHUNK
        ib = in_bounds[:, lo:hi]
        sk = jnp.zeros((1, _CHUNK), jnp.float32)
        ck = jnp.zeros((1, _CHUNK), jnp.float32)
        for b in range(_NBOX):
            pcx = p_ref[b, 0:1, lo:hi]
            pcy = p_ref[b, 1:2, lo:hi]
            pw = p_ref[b, 2:3, lo:hi]
            ph = p_ref[b, 3:4, lo:hi]
            tcx = t_ref[b, 0:1, lo:hi]
            tcy = t_ref[b, 1:2, lo:hi]
            tw = t_ref[b, 2:3, lo:hi]
            th = t_ref[b, 3:4, lo:hi]

            px1 = pcx - 0.5 * pw
            px2 = pcx + 0.5 * pw
            py1 = pcy - 0.5 * ph
            py2 = pcy + 0.5 * ph
            tx1 = tcx - 0.5 * tw
            tx2 = tcx + 0.5 * tw
            ty1 = tcy - 0.5 * th
            ty2 = tcy + 0.5 * th

            dx = jnp.maximum(
                jnp.minimum(px2, tx2) - jnp.maximum(px1, tx1), 0.0
            )
            dy = jnp.maximum(
                jnp.minimum(py2, ty2) - jnp.maximum(py1, ty1), 0.0
            )
            inter = dx * dy
            area_p = jnp.abs(pw * ph)
            area_t = jnp.abs(tw * th)
            iou = inter / (area_p + area_t - inter + 1e-6)

            valid = (tcx != -1.0) & ib
            sk = sk + jnp.where(valid, iou, 0.0)
            ck = ck + jnp.where(valid, 1.0, 0.0)
        sum_ref[0, :, lo:hi] += sk
        cnt_ref[0, :, lo:hi] += ck


def kernel(pred, truth):
    p3 = jnp.transpose(pred, (1, 2, 0))
    t3 = jnp.transpose(truth, (1, 2, 0))

    in_spec = pl.BlockSpec(
        (_NBOX, 4, _LBLK), lambda i, j: (0, 0, i * _JSTEPS + j)
    )
    out_spec = pl.BlockSpec((1, 1, _LBLK), lambda i, j: (i, 0, 0))
    sums, cnts = pl.pallas_call(
        _iou_body,
        grid=(_CORES, _JSTEPS),
        in_specs=[in_spec, in_spec],
        out_specs=[out_spec, out_spec],
        out_shape=[
            jax.ShapeDtypeStruct((_CORES, 1, _LBLK), jnp.float32),
            jax.ShapeDtypeStruct((_CORES, 1, _LBLK), jnp.float32),
        ],
        compiler_params=pltpu.CompilerParams(
            dimension_semantics=("parallel", "arbitrary"),
        ),
    )(p3, t3)

    total_iou = jnp.sum(sums)
    n_valid = jnp.sum(cnts)
    mean_iou = jnp.where(
        n_valid > 0, total_iou / jnp.maximum(n_valid, 1.0), 0.0
    )
    return mean_iou.reshape(1, 1)


# speedup vs baseline: 217.7502x; 1.0603x over previous
import jax
import jax.numpy as jnp
from jax.experimental import pallas as pl
from jax.experimental.pallas import tpu as pltpu

_B = 1_000_000
_NBOX = 6
_LBLK = 63488
_CORES = 2
_JSTEPS = 8


_CHUNK = 2048


def _iou_body(p_ref, t_ref, sum_ref, cnt_ref):
    i = pl.program_id(0)
    j = pl.program_id(1)

    lane = jax.lax.broadcasted_iota(jnp.int32, (1, _LBLK), 1)
    base = (i * _JSTEPS + j) * _LBLK
    in_bounds = (base + lane) < _B

    @pl.when(j == 0)
    def _():
        sum_ref[...] = jnp.zeros_like(sum_ref)
        cnt_ref[...] = jnp.zeros_like(cnt_ref)

    for k in range(_LBLK // _CHUNK):
        lo, hi = k * _CHUNK, (k + 1) * _CHUNK
        ib = in_bounds[:, lo:hi]
        sk = jnp.zeros((1, _CHUNK), jnp.float32)
        ck = jnp.zeros((1, _CHUNK), jnp.float32)
        for b in range(_NBOX):
            pcx = p_ref[b, 0:1, lo:hi]
            pcy = p_ref[b, 1:2, lo:hi]
            pw = p_ref[b, 2:3, lo:hi]
            ph = p_ref[b, 3:4, lo:hi]
            tcx = t_ref[b, 0:1, lo:hi]
            tcy = t_ref[b, 1:2, lo:hi]
            tw = t_ref[b, 2:3, lo:hi]
            th = t_ref[b, 3:4, lo:hi]

            px1 = pcx - 0.5 * pw
            px2 = pcx + 0.5 * pw
            py1 = pcy - 0.5 * ph
            py2 = pcy + 0.5 * ph
            tx1 = tcx - 0.5 * tw
            tx2 = tcx + 0.5 * tw
            ty1 = tcy - 0.5 * th
            ty2 = tcy + 0.5 * th

            dx = jnp.maximum(
                jnp.minimum(px2, tx2) - jnp.maximum(px1, tx1), 0.0
            )
            dy = jnp.maximum(
                jnp.minimum(py2, ty2) - jnp.maximum(py1, ty1), 0.0
            )
            inter = dx * dy
            area_p = jnp.abs(pw * ph)
            area_t = jnp.abs(tw * th)
            iou = inter / (area_p + area_t - inter + 1e-6)

            valid = (tcx != -1.0) & ib
            sk = sk + jnp.where(valid, iou, 0.0)
            ck = ck + jnp.where(valid, 1.0, 0.0)
        sum_ref[0, :, lo:hi] += sk
        cnt_ref[0, :, lo:hi] += ck


def kernel(pred, truth):
    p3 = jnp.transpose(pred, (1, 2, 0))
    t3 = jnp.transpose(truth, (1, 2, 0))

    in_spec = pl.BlockSpec(
        (_NBOX, 4, _LBLK), lambda i, j: (0, 0, i * _JSTEPS + j)
    )
    out_spec = pl.BlockSpec((1, 1, _LBLK), lambda i, j: (i, 0, 0))
    sums, cnts = pl.pallas_call(
        _iou_body,
        grid=(_CORES, _JSTEPS),
        in_specs=[in_spec, in_spec],
        out_specs=[out_spec, out_spec],
        out_shape=[
            jax.ShapeDtypeStruct((_CORES, 1, _LBLK), jnp.float32),
            jax.ShapeDtypeStruct((_CORES, 1, _LBLK), jnp.float32),
        ],
        compiler_params=pltpu.CompilerParams(
            dimension_semantics=("parallel", "arbitrary"),
        ),
    )(p3, t3)

    total_iou = jnp.sum(sums)
    n_valid = jnp.sum(cnts)
    mean_iou = jnp.where(
        n_valid > 0, total_iou / jnp.maximum(n_valid, 1.0), 0.0
    )
    return mean_iou.reshape(1, 1)
